# baseline (speedup 1.0000x reference)
import jax
import jax.numpy as jnp
from jax import lax
from jax.experimental import pallas as pl
from jax.experimental.pallas import tpu as pltpu
from jax.experimental.pallas import tpu_sc as plsc

_N = 100000
_D = 128
_NC = 2
_NS = 16
_NW = _NC * _NS
_RPW = _N // _NW
_CHUNK = 125
_NCHUNKS = _RPW // _CHUNK
_UNROLL = 5


def _stage1_body(u_hbm, loc_hbm, dist_hbm, idx_hbm, u_v, buf, resd, resi,
                 sem0, sem1):
    wid = lax.axis_index("s") * _NC + lax.axis_index("c")
    base = wid * _RPW

    pltpu.sync_copy(u_hbm, u_v)
    u_regs = [u_v[pl.ds(16 * j, 16)] for j in range(8)]
    sems = (sem0, sem1)

    lane = lax.iota(jnp.int32, 16)
    rots = [(lane + k) & 15 for k in (1, 2, 4, 8)]

    def hsum(v):
        for idx in rots:
            v = v + v[idx]
        return v

    def start(c, b):
        off = pl.multiple_of((base + c * _CHUNK) * _D, _D)
        pltpu.make_async_copy(
            loc_hbm.at[pl.ds(off, _CHUNK * _D)], buf.at[b], sems[b]
        ).start()

    def wait(b):
        pltpu.make_async_copy(
            loc_hbm.at[pl.ds(0, _CHUNK * _D)], buf.at[b], sems[b]
        ).wait()

    def process(b, row0, carry):
        bufb = buf.at[b]

        def body(k, carry):
            bd, bi = carry
            dists = []
            for t in range(_UNROLL):
                r = k * _UNROLL + t
                acc = None
                for j in range(8):
                    dif = bufb[pl.ds(r * _D + 16 * j, 16)] - u_regs[j]
                    sq = dif * dif
                    acc = sq if acc is None else acc + sq
                dists.append(hsum(acc))
            for t in range(_UNROLL):
                r = k * _UNROLL + t
                better = dists[t] < bd
                bd = jnp.where(better, dists[t], bd)
                bi = jnp.where(better, jnp.full((16,), row0 + r, jnp.int32), bi)
            return bd, bi

        return lax.fori_loop(0, _CHUNK // _UNROLL, body, carry)

    start(0, 0)
    init = (jnp.full((16,), jnp.inf, jnp.float32),
            jnp.zeros((16,), jnp.int32))

    def outer(g2, carry):
        c0 = 2 * g2
        start(c0 + 1, 1)
        wait(0)
        carry = process(0, base + c0 * _CHUNK, carry)
        start(c0 + 2, 0)
        wait(1)
        carry = process(1, base + (c0 + 1) * _CHUNK, carry)
        return carry

    carry = lax.fori_loop(0, (_NCHUNKS - 1) // 2, outer, init)
    wait(0)
    best_d, best_i = process(0, base + (_NCHUNKS - 1) * _CHUNK, carry)

    resd[...] = best_d
    resi[...] = best_i
    pltpu.sync_copy(resd, dist_hbm.at[pl.ds(wid * 16, 16)])
    pltpu.sync_copy(resi, idx_hbm.at[pl.ds(wid * 16, 16)])


def _stage2_body(dist_hbm, idx_hbm, loc_hbm, out_hbm, dbuf, ibuf, isel, rows,
                 sem):
    wid = lax.axis_index("s") * _NC + lax.axis_index("c")

    @pl.when(wid == 0)
    def _():
        pltpu.sync_copy(dist_hbm, dbuf)
        pltpu.sync_copy(idx_hbm, ibuf)
        acc_d = jnp.full((16,), jnp.inf, jnp.float32)
        acc_i = jnp.zeros((16,), jnp.int32)
        for w in range(_NW):
            dw = dbuf[pl.ds(w * 16, 16)]
            iw = ibuf[pl.ds(w * 16, 16)]
            m = dw < acc_d
            acc_d = jnp.where(m, dw, acc_d)
            acc_i = jnp.where(m, iw, acc_i)
        isel[...] = acc_i
        pltpu.async_copy(loc_hbm.at[isel], rows, sem).wait()
        pltpu.sync_copy(rows.at[0], out_hbm)


def kernel(u, loc, p):
    del p
    loc_flat = loc.reshape(-1)
    mesh = plsc.VectorSubcoreMesh(core_axis_name="c", subcore_axis_name="s")
    stage1 = pl.kernel(
        _stage1_body,
        out_type=[
            jax.ShapeDtypeStruct((_NW * 16,), jnp.float32),
            jax.ShapeDtypeStruct((_NW * 16,), jnp.int32),
        ],
        mesh=mesh,
        scratch_types=[
            pltpu.VMEM((_D,), jnp.float32),
            pltpu.VMEM((2, _CHUNK * _D), jnp.float32),
            pltpu.VMEM((16,), jnp.float32),
            pltpu.VMEM((16,), jnp.int32),
            pltpu.SemaphoreType.DMA,
            pltpu.SemaphoreType.DMA,
        ],
    )
    dists, idxs = stage1(u, loc_flat)
    stage2 = pl.kernel(
        _stage2_body,
        out_type=jax.ShapeDtypeStruct((_D,), jnp.float32),
        mesh=plsc.VectorSubcoreMesh(core_axis_name="c", subcore_axis_name="s"),
        scratch_types=[
            pltpu.VMEM((_NW * 16,), jnp.float32),
            pltpu.VMEM((_NW * 16,), jnp.int32),
            pltpu.VMEM((16,), jnp.int32),
            pltpu.VMEM((16, _D), jnp.float32),
            pltpu.SemaphoreType.DMA,
        ],
    )
    return stage2(dists, idxs, loc)
